# Initial kernel scaffold; baseline (speedup 1.0000x reference)
import jax
import jax.numpy as jnp
from jax.experimental import pallas as pl


def kernel(input, mempool):
    raise NotImplementedError("write your pallas kernel here")


# speedup vs baseline: 13.1310x; 13.1310x over previous
import functools

import jax
import jax.numpy as jnp
from jax.experimental import pallas as pl
from jax.experimental.pallas import tpu as pltpu

_N = 8192
_D = 64
_M = 8192
_K = 8
_BQ = 256


def _body(q_ref, mp_ref, o_ref):
    q = q_ref[...]
    mp = mp_ref[...]
    s = jax.lax.dot_general(q, mp, (((1,), (1,)), ((), ())),
                            preferred_element_type=jnp.float32)
    neg = jnp.float32(-3.0e38)
    r = s
    m0 = None
    mi = None
    for i in range(_K):
        mi = jnp.max(r, axis=1, keepdims=True)
        if i == 0:
            m0 = mi
        if i < _K - 1:
            r = jnp.where(r >= mi, neg, r)
    t8 = mi
    w = jnp.where(s >= t8, jnp.exp(s - m0), jnp.float32(0.0))
    z = jnp.sum(w, axis=1, keepdims=True)
    p = w / z
    o_ref[...] = jax.lax.dot_general(p, mp, (((1,), (0,)), ((), ())),
                                     preferred_element_type=jnp.float32)


@jax.jit
def kernel(input, mempool):
    B, C, H, W = input.shape
    q = jnp.transpose(input, (0, 2, 3, 1)).reshape(-1, C)
    out = pl.pallas_call(
        _body,
        grid=(_N // _BQ,),
        in_specs=[
            pl.BlockSpec((_BQ, _D), lambda i: (i, 0)),
            pl.BlockSpec((_M, _D), lambda i: (0, 0)),
        ],
        out_specs=pl.BlockSpec((_BQ, _D), lambda i: (i, 0)),
        out_shape=jax.ShapeDtypeStruct((_N, _D), jnp.float32),
    )(q, mempool)
    out = out.reshape(B, H, W, C)
    return jnp.transpose(out, (0, 3, 1, 2))


# speedup vs baseline: 13.3371x; 1.0157x over previous
import functools

import jax
import jax.numpy as jnp
from jax.experimental import pallas as pl
from jax.experimental.pallas import tpu as pltpu

_N = 8192
_D = 64
_M = 8192
_K = 8
_BQ = 256


def _body(q_ref, mp_ref, o_ref):
    q = q_ref[...]
    mp = mp_ref[...]
    s = jax.lax.dot_general(q, mp, (((1,), (1,)), ((), ())),
                            preferred_element_type=jnp.float32)
    neg = jnp.float32(-3.0e38)
    r = s
    m0 = None
    mi = None
    for i in range(_K):
        mi = jnp.max(r, axis=1, keepdims=True)
        if i == 0:
            m0 = mi
        if i < _K - 1:
            r = jnp.where(r >= mi, neg, r)
    t8 = mi
    w = jnp.where(s >= t8, jnp.exp(s - m0), jnp.float32(0.0))
    z = jnp.sum(w, axis=1, keepdims=True)
    o = jax.lax.dot_general(w, mp, (((1,), (0,)), ((), ())),
                            preferred_element_type=jnp.float32)
    o_ref[...] = o / z


@jax.jit
def kernel(input, mempool):
    B, C, H, W = input.shape
    q = jnp.transpose(input, (0, 2, 3, 1)).reshape(-1, C)
    out = pl.pallas_call(
        _body,
        grid=(_N // _BQ,),
        in_specs=[
            pl.BlockSpec((_BQ, _D), lambda i: (i, 0)),
            pl.BlockSpec((_M, _D), lambda i: (0, 0)),
        ],
        out_specs=pl.BlockSpec((_BQ, _D), lambda i: (i, 0)),
        out_shape=jax.ShapeDtypeStruct((_N, _D), jnp.float32),
    )(q, mempool)
    out = out.reshape(B, H, W, C)
    return jnp.transpose(out, (0, 3, 1, 2))


# speedup vs baseline: 14.0167x; 1.0510x over previous
import jax
import jax.numpy as jnp
from jax.experimental import pallas as pl
from jax.experimental.pallas import tpu as pltpu

_N = 8192
_D = 64
_M = 8192
_K = 8
_BQ = 256
_LANES = 128
_CHUNKS = _M // _LANES
_RC = 8
_NEG = -3.0e38


def _body(q_ref, mp_ref, o_ref, s_ref, cand_ref):
    q = q_ref[...]
    mp = mp_ref[...]
    s_ref[...] = jax.lax.dot_general(q, mp, (((1,), (1,)), ((), ())),
                                     preferred_element_type=jnp.float32)

    def row_chunk(i, carry):
        sl = s_ref[pl.ds(i * _RC, _RC), :]
        accs = [jnp.full((_RC, _LANES), _NEG, jnp.float32) for _ in range(_K)]
        for c in range(_CHUNKS):
            v = jax.lax.slice(sl, (0, c * _LANES), (_RC, (c + 1) * _LANES))
            for j in range(_K):
                hi = jnp.maximum(accs[j], v)
                if j < _K - 1:
                    v = jnp.minimum(accs[j], v)
                accs[j] = hi
        cand_ref[pl.ds(i * _RC, _RC), :] = jnp.concatenate(accs, axis=1)
        return carry

    jax.lax.fori_loop(0, _BQ // _RC, row_chunk, 0, unroll=False)

    r = cand_ref[...]
    m0 = None
    mi = None
    for i in range(_K):
        mi = jnp.max(r, axis=1, keepdims=True)
        if i == 0:
            m0 = mi
        if i < _K - 1:
            r = jnp.where(r >= mi, jnp.float32(_NEG), r)
    t8 = mi

    s = s_ref[...]
    w = jnp.where(s >= t8, jnp.exp(s - m0), jnp.float32(0.0))
    z = jnp.sum(w, axis=1, keepdims=True)
    o = jax.lax.dot_general(w, mp, (((1,), (0,)), ((), ())),
                            preferred_element_type=jnp.float32)
    o_ref[...] = o / z


@jax.jit
def kernel(input, mempool):
    B, C, H, W = input.shape
    q = jnp.transpose(input, (0, 2, 3, 1)).reshape(-1, C)
    out = pl.pallas_call(
        _body,
        grid=(_N // _BQ,),
        in_specs=[
            pl.BlockSpec((_BQ, _D), lambda i: (i, 0)),
            pl.BlockSpec((_M, _D), lambda i: (0, 0)),
        ],
        out_specs=pl.BlockSpec((_BQ, _D), lambda i: (i, 0)),
        out_shape=jax.ShapeDtypeStruct((_N, _D), jnp.float32),
        scratch_shapes=[
            pltpu.VMEM((_BQ, _M), jnp.float32),
            pltpu.VMEM((_BQ, _K * _LANES), jnp.float32),
        ],
    )(q, mempool)
    out = out.reshape(B, H, W, C)
    return jnp.transpose(out, (0, 3, 1, 2))


# speedup vs baseline: 14.2465x; 1.0164x over previous
import jax
import jax.numpy as jnp
from jax.experimental import pallas as pl
from jax.experimental.pallas import tpu as pltpu

_N = 8192
_D = 64
_M = 8192
_K = 8
_BQ = 256
_LANES = 128
_CHUNKS = _M // _LANES
_RC = 8
_NEG = -3.0e38


def _body(q_ref, mp_ref, o_ref, s_ref, cand_ref):
    q = q_ref[...]
    mp = mp_ref[...]
    s_ref[...] = jax.lax.dot_general(q, mp, (((1,), (1,)), ((), ())),
                                     preferred_element_type=jnp.float32)

    def row_chunk(i, carry):
        sl = s_ref[pl.ds(i * 4 * _RC, 4 * _RC), :]
        accs = [[jnp.full((_RC, _LANES), _NEG, jnp.float32) for _ in range(_K)]
                for _ in range(4)]
        for c in range(_CHUNKS):
            for g in range(4):
                v = jax.lax.slice(sl, (g * _RC, c * _LANES),
                                  ((g + 1) * _RC, (c + 1) * _LANES))
                for j in range(_K):
                    hi = jnp.maximum(accs[g][j], v)
                    if j < _K - 1:
                        v = jnp.minimum(accs[g][j], v)
                    accs[g][j] = hi
        cand_ref[pl.ds(i * 4 * _RC, 4 * _RC), :] = jnp.concatenate(
            [jnp.concatenate(a, axis=1) for a in accs], axis=0)
        return carry

    jax.lax.fori_loop(0, _BQ // (4 * _RC), row_chunk, 0, unroll=False)

    r = cand_ref[...]
    m0 = None
    mi = None
    for i in range(_K):
        mi = jnp.max(r, axis=1, keepdims=True)
        if i == 0:
            m0 = mi
        if i < _K - 1:
            r = jnp.where(r >= mi, jnp.float32(_NEG), r)
    t8 = mi

    s = s_ref[...]
    w = jnp.where(s >= t8, jnp.exp(s - m0), jnp.float32(0.0))
    z = jnp.sum(w, axis=1, keepdims=True)
    o = jax.lax.dot_general(w, mp, (((1,), (0,)), ((), ())),
                            preferred_element_type=jnp.float32)
    o_ref[...] = o / z


@jax.jit
def kernel(input, mempool):
    B, C, H, W = input.shape
    q = jnp.transpose(input, (0, 2, 3, 1)).reshape(-1, C)
    out = pl.pallas_call(
        _body,
        grid=(_N // _BQ,),
        in_specs=[
            pl.BlockSpec((_BQ, _D), lambda i: (i, 0)),
            pl.BlockSpec((_M, _D), lambda i: (0, 0)),
        ],
        out_specs=pl.BlockSpec((_BQ, _D), lambda i: (i, 0)),
        out_shape=jax.ShapeDtypeStruct((_N, _D), jnp.float32),
        scratch_shapes=[
            pltpu.VMEM((_BQ, _M), jnp.float32),
            pltpu.VMEM((_BQ, _K * _LANES), jnp.float32),
        ],
    )(q, mempool)
    out = out.reshape(B, H, W, C)
    return jnp.transpose(out, (0, 3, 1, 2))


# speedup vs baseline: 18.4504x; 1.2951x over previous
import jax
import jax.numpy as jnp
from jax.experimental import pallas as pl
from jax.experimental.pallas import tpu as pltpu

_N = 8192
_D = 64
_M = 8192
_K = 8
_BQ = 256
_LANES = 128
_CHUNKS = _M // _LANES
_RC = 8
_NEG = -3.0e38


def _body(q_ref, mp_ref, o_ref, s_ref, cand_ref):
    q = q_ref[...]
    mp = mp_ref[...]
    s_ref[...] = jax.lax.dot_general(q, mp, (((1,), (1,)), ((), ())),
                                     preferred_element_type=jnp.float32)

    sort8_pairs = ((0, 1), (2, 3), (4, 5), (6, 7),
                   (0, 2), (1, 3), (4, 6), (5, 7),
                   (1, 2), (5, 6),
                   (0, 4), (1, 5), (2, 6), (3, 7),
                   (2, 4), (3, 5),
                   (1, 2), (3, 4), (5, 6))
    bitonic_pairs = ((0, 4), (1, 5), (2, 6), (3, 7),
                     (0, 2), (1, 3), (4, 6), (5, 7),
                     (0, 1), (2, 3), (4, 5), (6, 7))

    def _sort8(vs):
        for a, b in sort8_pairs:
            hi = jnp.maximum(vs[a], vs[b])
            lo = jnp.minimum(vs[a], vs[b])
            vs[a], vs[b] = hi, lo
        return vs

    def _merge8(x, y, sort_out=True):
        t = [jnp.maximum(x[i], y[7 - i]) for i in range(8)]
        if sort_out:
            for a, b in bitonic_pairs:
                hi = jnp.maximum(t[a], t[b])
                lo = jnp.minimum(t[a], t[b])
                t[a], t[b] = hi, lo
        return t

    def row_chunk(i, carry):
        sl = s_ref[pl.ds(i * _RC, _RC), :]
        def srt(g):
            vs = [jax.lax.slice(sl, (0, (8 * g + c) * _LANES),
                                (_RC, (8 * g + c + 1) * _LANES))
                  for c in range(8)]
            return _sort8(vs)

        m03 = _merge8(_merge8(srt(0), srt(1)), _merge8(srt(2), srt(3)))
        m47 = _merge8(_merge8(srt(4), srt(5)), _merge8(srt(6), srt(7)))
        top = _merge8(m03, m47, sort_out=False)
        cand_ref[pl.ds(i * _RC, _RC), :] = jnp.concatenate(top, axis=1)
        return carry

    jax.lax.fori_loop(0, _BQ // _RC, row_chunk, 0, unroll=False)

    r = cand_ref[...]
    mi = None
    for i in range(_K):
        mi = jnp.max(r, axis=1, keepdims=True)
        if i < _K - 1:
            r = jnp.where(r >= mi, jnp.float32(_NEG), r)
    t8 = mi

    s = s_ref[...]
    w = jnp.where(s >= t8, jnp.exp(s), jnp.float32(0.0))
    z = jnp.sum(w, axis=1, keepdims=True)
    o = jax.lax.dot_general(w, mp, (((1,), (0,)), ((), ())),
                            preferred_element_type=jnp.float32)
    o_ref[...] = o / z


@jax.jit
def kernel(input, mempool):
    B, C, H, W = input.shape
    q = jnp.transpose(input, (0, 2, 3, 1)).reshape(-1, C)
    out = pl.pallas_call(
        _body,
        grid=(_N // _BQ,),
        in_specs=[
            pl.BlockSpec((_BQ, _D), lambda i: (i, 0)),
            pl.BlockSpec((_M, _D), lambda i: (0, 0)),
        ],
        out_specs=pl.BlockSpec((_BQ, _D), lambda i: (i, 0)),
        out_shape=jax.ShapeDtypeStruct((_N, _D), jnp.float32),
        scratch_shapes=[
            pltpu.VMEM((_BQ, _M), jnp.float32),
            pltpu.VMEM((_BQ, _K * _LANES), jnp.float32),
        ],
    )(q, mempool)
    out = out.reshape(B, H, W, C)
    return jnp.transpose(out, (0, 3, 1, 2))


# speedup vs baseline: 21.4641x; 1.1633x over previous
import jax
import jax.numpy as jnp
from jax.experimental import pallas as pl
from jax.experimental.pallas import tpu as pltpu

_N = 8192
_D = 64
_M = 8192
_K = 8
_BQ = 256
_LANES = 128
_CHUNKS = _M // _LANES
_RC = 8
_NEG = -3.0e38

_SORT8 = ((0, 1), (2, 3), (4, 5), (6, 7),
          (0, 2), (1, 3), (4, 6), (5, 7),
          (1, 2), (5, 6),
          (0, 4), (1, 5), (2, 6), (3, 7),
          (2, 4), (3, 5),
          (1, 2), (3, 4), (5, 6))
_BITONIC = ((0, 4), (1, 5), (2, 6), (3, 7),
            (0, 2), (1, 3), (4, 6), (5, 7),
            (0, 1), (2, 3), (4, 5), (6, 7))


def _sort8(vs):
    for a, b in _SORT8:
        hi = jnp.maximum(vs[a], vs[b])
        lo = jnp.minimum(vs[a], vs[b])
        vs[a], vs[b] = hi, lo
    return vs


def _merge8(x, y, sort_out=True):
    t = [jnp.maximum(x[i], y[7 - i]) for i in range(8)]
    if sort_out:
        for a, b in _BITONIC:
            hi = jnp.maximum(t[a], t[b])
            lo = jnp.minimum(t[a], t[b])
            t[a], t[b] = hi, lo
    return t


def _process(s_ref, cand_ref, mp, o_ref):
    def row_chunk(i, carry):
        sl = s_ref[pl.ds(i * _RC, _RC), :]

        def srt(g):
            vs = [jax.lax.slice(sl, (0, (8 * g + c) * _LANES),
                                (_RC, (8 * g + c + 1) * _LANES))
                  for c in range(8)]
            return _sort8(vs)

        m03 = _merge8(_merge8(srt(0), srt(1)), _merge8(srt(2), srt(3)))
        m47 = _merge8(_merge8(srt(4), srt(5)), _merge8(srt(6), srt(7)))
        top = _merge8(m03, m47, sort_out=False)
        cand_ref[pl.ds(i * _RC, _RC), :] = jnp.concatenate(top, axis=1)
        return carry

    jax.lax.fori_loop(0, _BQ // _RC, row_chunk, 0, unroll=False)

    r = cand_ref[...]
    mi = None
    for i in range(_K):
        mi = jnp.max(r, axis=1, keepdims=True)
        if i < _K - 1:
            r = jnp.where(r >= mi, jnp.float32(_NEG), r)
    t8 = mi

    s = s_ref[...]
    w = jnp.where(s >= t8, jnp.exp(s), jnp.float32(0.0))
    z = jnp.sum(w, axis=1, keepdims=True)
    o = jax.lax.dot_general(w, mp, (((1,), (0,)), ((), ())),
                            preferred_element_type=jnp.float32)
    o_ref[...] = o / z


def _body(q_ref, mp_ref, o_ref, sa_ref, sb_ref, cand_ref):
    i = pl.program_id(0)
    par = jax.lax.rem(i, 2)
    q = q_ref[...]
    mp = mp_ref[...]
    steps = pl.num_programs(0)

    @pl.when(par == 0)
    def _even():
        @pl.when(i < steps - 1)
        def _():
            sa_ref[...] = jax.lax.dot_general(
                q, mp, (((1,), (1,)), ((), ())),
                preferred_element_type=jnp.float32)
        _process(sb_ref, cand_ref, mp, o_ref)

    @pl.when(par == 1)
    def _odd():
        @pl.when(i < steps - 1)
        def _():
            sb_ref[...] = jax.lax.dot_general(
                q, mp, (((1,), (1,)), ((), ())),
                preferred_element_type=jnp.float32)
        _process(sa_ref, cand_ref, mp, o_ref)


@jax.jit
def kernel(input, mempool):
    B, C, H, W = input.shape
    q = jnp.transpose(input, (0, 2, 3, 1)).reshape(-1, C)
    nblk = _N // _BQ
    out = pl.pallas_call(
        _body,
        grid=(nblk + 1,),
        in_specs=[
            pl.BlockSpec((_BQ, _D), lambda i: (jnp.minimum(i, nblk - 1), 0)),
            pl.BlockSpec((_M, _D), lambda i: (0, 0)),
        ],
        out_specs=pl.BlockSpec((_BQ, _D), lambda i: (jnp.maximum(i - 1, 0), 0)),
        out_shape=jax.ShapeDtypeStruct((_N, _D), jnp.float32),
        scratch_shapes=[
            pltpu.VMEM((_BQ, _M), jnp.float32),
            pltpu.VMEM((_BQ, _M), jnp.float32),
            pltpu.VMEM((_BQ, _K * _LANES), jnp.float32),
        ],
    )(q, mempool)
    out = out.reshape(B, H, W, C)
    return jnp.transpose(out, (0, 3, 1, 2))


# speedup vs baseline: 21.6392x; 1.0082x over previous
import jax
import jax.numpy as jnp
from jax.experimental import pallas as pl
from jax.experimental.pallas import tpu as pltpu

_N = 8192
_D = 64
_M = 8192
_K = 8
_BQ = 256
_LANES = 128
_CHUNKS = _M // _LANES
_RC = 8
_NEG = -3.0e38

_SORT8 = ((0, 1), (2, 3), (4, 5), (6, 7),
          (0, 2), (1, 3), (4, 6), (5, 7),
          (1, 2), (5, 6),
          (0, 4), (1, 5), (2, 6), (3, 7),
          (2, 4), (3, 5),
          (1, 2), (3, 4), (5, 6))
_BITONIC = ((0, 4), (1, 5), (2, 6), (3, 7),
            (0, 2), (1, 3), (4, 6), (5, 7),
            (0, 1), (2, 3), (4, 5), (6, 7))


def _sort8(vs):
    for a, b in _SORT8:
        hi = jnp.maximum(vs[a], vs[b])
        lo = jnp.minimum(vs[a], vs[b])
        vs[a], vs[b] = hi, lo
    return vs


def _merge8(x, y, sort_out=True):
    t = [jnp.maximum(x[i], y[7 - i]) for i in range(8)]
    if sort_out:
        for a, b in _BITONIC:
            hi = jnp.maximum(t[a], t[b])
            lo = jnp.minimum(t[a], t[b])
            t[a], t[b] = hi, lo
    return t


def _process(s_ref, cand_ref, mp, o_ref):
    def row_chunk(i, carry):
        sl = s_ref[pl.ds(i * _RC, _RC), :]

        def srt(g):
            vs = [jax.lax.slice(sl, (0, (8 * g + c) * _LANES),
                                (_RC, (8 * g + c + 1) * _LANES))
                  for c in range(8)]
            return _sort8(vs)

        m03 = _merge8(_merge8(srt(0), srt(1)), _merge8(srt(2), srt(3)))
        m47 = _merge8(_merge8(srt(4), srt(5)), _merge8(srt(6), srt(7)))
        top = _merge8(m03, m47, sort_out=False)
        cand_ref[pl.ds(i * _RC, _RC), :] = jnp.concatenate(top, axis=1)
        return carry

    jax.lax.fori_loop(0, _BQ // _RC, row_chunk, 0, unroll=2)

    r = cand_ref[...]
    mi = None
    for i in range(_K):
        mi = jnp.max(r, axis=1, keepdims=True)
        if i < _K - 1:
            r = jnp.where(r >= mi, jnp.float32(_NEG), r)
    t8 = mi

    s = s_ref[...]
    w = jnp.where(s >= t8, jnp.exp2(s * jnp.float32(1.4426950408889634)),
                  jnp.float32(0.0))
    z = jnp.sum(w, axis=1, keepdims=True)
    o = jax.lax.dot_general(w, mp, (((1,), (0,)), ((), ())),
                            preferred_element_type=jnp.float32)
    o_ref[...] = o / z


def _body(q_ref, mp_ref, o_ref, sa_ref, sb_ref, cand_ref):
    i = pl.program_id(0)
    par = jax.lax.rem(i, 2)
    q = q_ref[...]
    mp = mp_ref[...]
    steps = pl.num_programs(0)

    @pl.when(par == 0)
    def _even():
        @pl.when(i < steps - 1)
        def _():
            sa_ref[...] = jax.lax.dot_general(
                q, mp, (((1,), (1,)), ((), ())),
                preferred_element_type=jnp.float32)
        _process(sb_ref, cand_ref, mp, o_ref)

    @pl.when(par == 1)
    def _odd():
        @pl.when(i < steps - 1)
        def _():
            sb_ref[...] = jax.lax.dot_general(
                q, mp, (((1,), (1,)), ((), ())),
                preferred_element_type=jnp.float32)
        _process(sa_ref, cand_ref, mp, o_ref)


@jax.jit
def kernel(input, mempool):
    B, C, H, W = input.shape
    q = jnp.transpose(input, (0, 2, 3, 1)).reshape(-1, C)
    nblk = _N // _BQ
    out = pl.pallas_call(
        _body,
        grid=(nblk + 1,),
        in_specs=[
            pl.BlockSpec((_BQ, _D), lambda i: (jnp.minimum(i, nblk - 1), 0)),
            pl.BlockSpec((_M, _D), lambda i: (0, 0)),
        ],
        out_specs=pl.BlockSpec((_BQ, _D), lambda i: (jnp.maximum(i - 1, 0), 0)),
        out_shape=jax.ShapeDtypeStruct((_N, _D), jnp.float32),
        scratch_shapes=[
            pltpu.VMEM((_BQ, _M), jnp.float32),
            pltpu.VMEM((_BQ, _M), jnp.float32),
            pltpu.VMEM((_BQ, _K * _LANES), jnp.float32),
        ],
    )(q, mempool)
    out = out.reshape(B, H, W, C)
    return jnp.transpose(out, (0, 3, 1, 2))


# speedup vs baseline: 21.6446x; 1.0003x over previous
import jax
import jax.numpy as jnp
from jax.experimental import pallas as pl
from jax.experimental.pallas import tpu as pltpu

_N = 8192
_D = 64
_M = 8192
_K = 8
_BQ = 256
_LANES = 128
_CHUNKS = _M // _LANES
_RC = 16
_NEG = -3.0e38

_SORT8 = ((0, 1), (2, 3), (4, 5), (6, 7),
          (0, 2), (1, 3), (4, 6), (5, 7),
          (1, 2), (5, 6),
          (0, 4), (1, 5), (2, 6), (3, 7),
          (2, 4), (3, 5),
          (1, 2), (3, 4), (5, 6))
_BITONIC = ((0, 4), (1, 5), (2, 6), (3, 7),
            (0, 2), (1, 3), (4, 6), (5, 7),
            (0, 1), (2, 3), (4, 5), (6, 7))


def _sort8(vs):
    for a, b in _SORT8:
        hi = jnp.maximum(vs[a], vs[b])
        lo = jnp.minimum(vs[a], vs[b])
        vs[a], vs[b] = hi, lo
    return vs


def _merge8(x, y, sort_out=True):
    t = [jnp.maximum(x[i], y[7 - i]) for i in range(8)]
    if sort_out:
        for a, b in _BITONIC:
            hi = jnp.maximum(t[a], t[b])
            lo = jnp.minimum(t[a], t[b])
            t[a], t[b] = hi, lo
    return t


def _process(s_ref, cand_ref, mp, o_ref):
    def row_chunk(i, carry):
        sl = s_ref[pl.ds(i * _RC, _RC), :]

        def srt(g):
            vs = [jax.lax.slice(sl, (0, (8 * g + c) * _LANES),
                                (_RC, (8 * g + c + 1) * _LANES))
                  for c in range(8)]
            return _sort8(vs)

        m03 = _merge8(_merge8(srt(0), srt(1)), _merge8(srt(2), srt(3)))
        m47 = _merge8(_merge8(srt(4), srt(5)), _merge8(srt(6), srt(7)))
        top = _merge8(m03, m47, sort_out=False)
        cand_ref[pl.ds(i * _RC, _RC), :] = jnp.concatenate(top, axis=1)
        return carry

    jax.lax.fori_loop(0, _BQ // _RC, row_chunk, 0, unroll=False)

    r = cand_ref[...]
    mi = None
    for i in range(_K):
        mi = jnp.max(r, axis=1, keepdims=True)
        if i < _K - 1:
            r = jnp.where(r >= mi, jnp.float32(_NEG), r)
    t8 = mi

    s = s_ref[...]
    w = jnp.where(s >= t8, jnp.exp2(s * jnp.float32(1.4426950408889634)),
                  jnp.float32(0.0))
    z = jnp.sum(w, axis=1, keepdims=True)
    o = jax.lax.dot_general(w, mp, (((1,), (0,)), ((), ())),
                            preferred_element_type=jnp.float32)
    o_ref[...] = o / z


def _body(q_ref, mp_ref, o_ref, sa_ref, sb_ref, cand_ref):
    i = pl.program_id(0)
    par = jax.lax.rem(i, 2)
    q = q_ref[...]
    mp = mp_ref[...]
    steps = pl.num_programs(0)

    @pl.when(par == 0)
    def _even():
        @pl.when(i < steps - 1)
        def _():
            sa_ref[...] = jax.lax.dot_general(
                q, mp, (((1,), (1,)), ((), ())),
                preferred_element_type=jnp.float32)
        _process(sb_ref, cand_ref, mp, o_ref)

    @pl.when(par == 1)
    def _odd():
        @pl.when(i < steps - 1)
        def _():
            sb_ref[...] = jax.lax.dot_general(
                q, mp, (((1,), (1,)), ((), ())),
                preferred_element_type=jnp.float32)
        _process(sa_ref, cand_ref, mp, o_ref)


@jax.jit
def kernel(input, mempool):
    B, C, H, W = input.shape
    q = jnp.transpose(input, (0, 2, 3, 1)).reshape(-1, C)
    nblk = _N // _BQ
    out = pl.pallas_call(
        _body,
        grid=(nblk + 1,),
        in_specs=[
            pl.BlockSpec((_BQ, _D), lambda i: (jnp.minimum(i, nblk - 1), 0)),
            pl.BlockSpec((_M, _D), lambda i: (0, 0)),
        ],
        out_specs=pl.BlockSpec((_BQ, _D), lambda i: (jnp.maximum(i - 1, 0), 0)),
        out_shape=jax.ShapeDtypeStruct((_N, _D), jnp.float32),
        scratch_shapes=[
            pltpu.VMEM((_BQ, _M), jnp.float32),
            pltpu.VMEM((_BQ, _M), jnp.float32),
            pltpu.VMEM((_BQ, _K * _LANES), jnp.float32),
        ],
    )(q, mempool)
    out = out.reshape(B, H, W, C)
    return jnp.transpose(out, (0, 3, 1, 2))


# speedup vs baseline: 25.8888x; 1.1961x over previous
import jax
import jax.numpy as jnp
from jax.experimental import pallas as pl
from jax.experimental.pallas import tpu as pltpu

_N = 8192
_D = 64
_M = 8192
_K = 8
_BQ = 256
_LANES = 128
_CHUNKS = _M // _LANES
_RC = 16
_NEG = -3.0e38

_SORT8 = ((0, 1), (2, 3), (4, 5), (6, 7),
          (0, 2), (1, 3), (4, 6), (5, 7),
          (1, 2), (5, 6),
          (0, 4), (1, 5), (2, 6), (3, 7),
          (2, 4), (3, 5),
          (1, 2), (3, 4), (5, 6))
_BITONIC = ((0, 4), (1, 5), (2, 6), (3, 7),
            (0, 2), (1, 3), (4, 6), (5, 7),
            (0, 1), (2, 3), (4, 5), (6, 7))


def _sort8(vs):
    for a, b in _SORT8:
        hi = jnp.maximum(vs[a], vs[b])
        lo = jnp.minimum(vs[a], vs[b])
        vs[a], vs[b] = hi, lo
    return vs


def _merge8(x, y, sort_out=True):
    t = [jnp.maximum(x[i], y[7 - i]) for i in range(8)]
    if sort_out:
        for a, b in _BITONIC:
            hi = jnp.maximum(t[a], t[b])
            lo = jnp.minimum(t[a], t[b])
            t[a], t[b] = hi, lo
    return t


def _process(s_ref, cand_ref, mp, o_ref):
    def row_chunk(i, carry):
        sl = s_ref[pl.ds(i * _RC, _RC), :]

        def srt(g):
            vs = [jax.lax.slice(sl, (0, (8 * g + c) * _LANES),
                                (_RC, (8 * g + c + 1) * _LANES))
                  for c in range(8)]
            return _sort8(vs)

        m03 = _merge8(_merge8(srt(0), srt(1)), _merge8(srt(2), srt(3)))
        m47 = _merge8(_merge8(srt(4), srt(5)), _merge8(srt(6), srt(7)))
        top = _merge8(m03, m47, sort_out=False)
        cand_ref[pl.ds(i * _RC, _RC), :] = jnp.concatenate(top, axis=1)
        return carry

    jax.lax.fori_loop(0, _BQ // _RC, row_chunk, 0, unroll=False)

    r = cand_ref[...]
    mi = None
    for i in range(_K):
        mi = jnp.max(r, axis=1, keepdims=True)
        if i < _K - 1:
            r = jnp.where(r >= mi, jnp.float32(_NEG), r)
    t8 = mi

    c = cand_ref[...]
    log2e = jnp.float32(1.4426950408889634)
    wc = jnp.where(c >= t8, jnp.exp2(c * log2e), jnp.float32(0.0))
    z = jnp.sum(wc, axis=1, keepdims=True)

    s = s_ref[...]
    w = jnp.where(s >= t8, jnp.exp2(s * log2e), jnp.float32(0.0))
    o = jax.lax.dot_general(w, mp, (((1,), (0,)), ((), ())),
                            preferred_element_type=jnp.float32)
    o_ref[...] = o / z


def _body(q_ref, mp_ref, o_ref, sa_ref, sb_ref, cand_ref):
    i = pl.program_id(0)
    par = jax.lax.rem(i, 2)
    q = q_ref[...]
    mp = mp_ref[...]
    steps = pl.num_programs(0)

    @pl.when(par == 0)
    def _even():
        @pl.when(i < steps - 1)
        def _():
            sa_ref[...] = jax.lax.dot_general(
                q, mp, (((1,), (1,)), ((), ())),
                preferred_element_type=jnp.float32)
        _process(sb_ref, cand_ref, mp, o_ref)

    @pl.when(par == 1)
    def _odd():
        @pl.when(i < steps - 1)
        def _():
            sb_ref[...] = jax.lax.dot_general(
                q, mp, (((1,), (1,)), ((), ())),
                preferred_element_type=jnp.float32)
        _process(sa_ref, cand_ref, mp, o_ref)


@jax.jit
def kernel(input, mempool):
    B, C, H, W = input.shape
    q = jnp.transpose(input, (0, 2, 3, 1)).reshape(-1, C)
    nblk = _N // _BQ
    out = pl.pallas_call(
        _body,
        grid=(nblk + 1,),
        in_specs=[
            pl.BlockSpec((_BQ, _D), lambda i: (jnp.minimum(i, nblk - 1), 0)),
            pl.BlockSpec((_M, _D), lambda i: (0, 0)),
        ],
        out_specs=pl.BlockSpec((_BQ, _D), lambda i: (jnp.maximum(i - 1, 0), 0)),
        out_shape=jax.ShapeDtypeStruct((_N, _D), jnp.float32),
        scratch_shapes=[
            pltpu.VMEM((_BQ, _M), jnp.float32),
            pltpu.VMEM((_BQ, _M), jnp.float32),
            pltpu.VMEM((_BQ, _K * _LANES), jnp.float32),
        ],
    )(q, mempool)
    out = out.reshape(B, H, W, C)
    return jnp.transpose(out, (0, 3, 1, 2))


# speedup vs baseline: 26.0595x; 1.0066x over previous
import jax
import jax.numpy as jnp
from jax.experimental import pallas as pl
from jax.experimental.pallas import tpu as pltpu

_N = 8192
_D = 64
_M = 8192
_K = 8
_BQ = 256
_LANES = 128
_CHUNKS = _M // _LANES
_RC = 16
_NEG = -3.0e38

_SORT8 = ((0, 1), (2, 3), (4, 5), (6, 7),
          (0, 2), (1, 3), (4, 6), (5, 7),
          (1, 2), (5, 6),
          (0, 4), (1, 5), (2, 6), (3, 7),
          (2, 4), (3, 5),
          (1, 2), (3, 4), (5, 6))
_BITONIC = ((0, 4), (1, 5), (2, 6), (3, 7),
            (0, 2), (1, 3), (4, 6), (5, 7),
            (0, 1), (2, 3), (4, 5), (6, 7))


def _sort8(vs):
    for a, b in _SORT8:
        hi = jnp.maximum(vs[a], vs[b])
        lo = jnp.minimum(vs[a], vs[b])
        vs[a], vs[b] = hi, lo
    return vs


def _merge8(x, y, sort_out=True):
    t = [jnp.maximum(x[i], y[7 - i]) for i in range(8)]
    if sort_out:
        for a, b in _BITONIC:
            hi = jnp.maximum(t[a], t[b])
            lo = jnp.minimum(t[a], t[b])
            t[a], t[b] = hi, lo
    return t


def _process(s_ref, cand_ref, mp, o_ref):
    def row_chunk(i, carry):
        sl = s_ref[pl.ds(i * _RC, _RC), :]

        def srt(g):
            vs = [jax.lax.slice(sl, (0, (8 * g + c) * _LANES),
                                (_RC, (8 * g + c + 1) * _LANES))
                  for c in range(8)]
            return _sort8(vs)

        m03 = _merge8(_merge8(srt(0), srt(1)), _merge8(srt(2), srt(3)))
        m47 = _merge8(_merge8(srt(4), srt(5)), _merge8(srt(6), srt(7)))
        top = _merge8(m03, m47, sort_out=False)
        cand_ref[pl.ds(i * _RC, _RC), :] = jnp.concatenate(top, axis=1)
        return carry

    jax.lax.fori_loop(0, _BQ // _RC, row_chunk, 0, unroll=2)

    r = cand_ref[...]
    mi = None
    for i in range(_K):
        mi = jnp.max(r, axis=1, keepdims=True)
        if i < _K - 1:
            r = jnp.where(r >= mi, jnp.float32(_NEG), r)
    t8 = mi

    c = cand_ref[...]
    log2e = jnp.float32(1.4426950408889634)
    wc = jnp.where(c >= t8, jnp.exp2(c * log2e), jnp.float32(0.0))
    z = jnp.sum(wc, axis=1, keepdims=True)

    s = s_ref[...]
    w = jnp.where(s >= t8, jnp.exp2(s * log2e), jnp.float32(0.0))
    o = jax.lax.dot_general(w, mp, (((1,), (0,)), ((), ())),
                            preferred_element_type=jnp.float32)
    o_ref[...] = o / z


def _body(q_ref, mp_ref, o_ref, sa_ref, sb_ref, cand_ref):
    i = pl.program_id(0)
    par = jax.lax.rem(i, 2)
    q = q_ref[...]
    mp = mp_ref[...]
    steps = pl.num_programs(0)

    @pl.when(par == 0)
    def _even():
        @pl.when(i < steps - 1)
        def _():
            sa_ref[...] = jax.lax.dot_general(
                q, mp, (((1,), (1,)), ((), ())),
                preferred_element_type=jnp.float32)
        _process(sb_ref, cand_ref, mp, o_ref)

    @pl.when(par == 1)
    def _odd():
        @pl.when(i < steps - 1)
        def _():
            sb_ref[...] = jax.lax.dot_general(
                q, mp, (((1,), (1,)), ((), ())),
                preferred_element_type=jnp.float32)
        _process(sa_ref, cand_ref, mp, o_ref)


@jax.jit
def kernel(input, mempool):
    B, C, H, W = input.shape
    q = jnp.transpose(input, (0, 2, 3, 1)).reshape(-1, C)
    nblk = _N // _BQ
    out = pl.pallas_call(
        _body,
        grid=(nblk + 1,),
        in_specs=[
            pl.BlockSpec((_BQ, _D), lambda i: (jnp.minimum(i, nblk - 1), 0)),
            pl.BlockSpec((_M, _D), lambda i: (0, 0)),
        ],
        out_specs=pl.BlockSpec((_BQ, _D), lambda i: (jnp.maximum(i - 1, 0), 0)),
        out_shape=jax.ShapeDtypeStruct((_N, _D), jnp.float32),
        scratch_shapes=[
            pltpu.VMEM((_BQ, _M), jnp.float32),
            pltpu.VMEM((_BQ, _M), jnp.float32),
            pltpu.VMEM((_BQ, _K * _LANES), jnp.float32),
        ],
    )(q, mempool)
    out = out.reshape(B, H, W, C)
    return jnp.transpose(out, (0, 3, 1, 2))


# speedup vs baseline: 26.1158x; 1.0022x over previous
import jax
import jax.numpy as jnp
from jax.experimental import pallas as pl
from jax.experimental.pallas import tpu as pltpu

_N = 8192
_D = 64
_M = 8192
_K = 8
_BQ = 256
_LANES = 128
_CHUNKS = _M // _LANES
_RC = 16
_NEG = -3.0e38

_SORT8 = ((0, 1), (2, 3), (4, 5), (6, 7),
          (0, 2), (1, 3), (4, 6), (5, 7),
          (1, 2), (5, 6),
          (0, 4), (1, 5), (2, 6), (3, 7),
          (2, 4), (3, 5),
          (1, 2), (3, 4), (5, 6))
_BITONIC = ((0, 4), (1, 5), (2, 6), (3, 7),
            (0, 2), (1, 3), (4, 6), (5, 7),
            (0, 1), (2, 3), (4, 5), (6, 7))


def _sort8(vs):
    for a, b in _SORT8:
        hi = jnp.maximum(vs[a], vs[b])
        lo = jnp.minimum(vs[a], vs[b])
        vs[a], vs[b] = hi, lo
    return vs


def _merge8(x, y, sort_out=True):
    t = [jnp.maximum(x[i], y[7 - i]) for i in range(8)]
    if sort_out:
        for a, b in _BITONIC:
            hi = jnp.maximum(t[a], t[b])
            lo = jnp.minimum(t[a], t[b])
            t[a], t[b] = hi, lo
    return t


def _process(s_ref, cand_ref, mp, o_ref):
    def row_chunk(i, carry):
        sl = s_ref[pl.ds(i * _RC, _RC), :]

        def srt(g):
            vs = [jax.lax.slice(sl, (0, (8 * g + c) * _LANES),
                                (_RC, (8 * g + c + 1) * _LANES))
                  for c in range(8)]
            return _sort8(vs)

        m03 = _merge8(_merge8(srt(0), srt(1)), _merge8(srt(2), srt(3)))
        m47 = _merge8(_merge8(srt(4), srt(5)), _merge8(srt(6), srt(7)))
        top = _merge8(m03, m47, sort_out=False)
        cand_ref[pl.ds(i * _RC, _RC), :] = jnp.concatenate(top, axis=1)
        return carry

    jax.lax.fori_loop(0, _BQ // _RC, row_chunk, 0, unroll=4)

    r = cand_ref[...]
    mi = None
    for i in range(_K):
        mi = jnp.max(r, axis=1, keepdims=True)
        if i < _K - 1:
            r = jnp.where(r >= mi, jnp.float32(_NEG), r)
    t8 = mi

    c = cand_ref[...]
    log2e = jnp.float32(1.4426950408889634)
    wc = jnp.where(c >= t8, jnp.exp2(c * log2e), jnp.float32(0.0))
    z = jnp.sum(wc, axis=1, keepdims=True)

    s = s_ref[...]
    w = jnp.where(s >= t8, jnp.exp2(s * log2e), jnp.float32(0.0))
    o = jax.lax.dot_general(w, mp, (((1,), (0,)), ((), ())),
                            preferred_element_type=jnp.float32)
    o_ref[...] = o / z


def _body(q_ref, mp_ref, o_ref, sa_ref, sb_ref, cand_ref):
    i = pl.program_id(0)
    par = jax.lax.rem(i, 2)
    q = q_ref[...]
    mp = mp_ref[...]
    steps = pl.num_programs(0)

    @pl.when(par == 0)
    def _even():
        @pl.when(i < steps - 1)
        def _():
            sa_ref[...] = jax.lax.dot_general(
                q, mp, (((1,), (1,)), ((), ())),
                preferred_element_type=jnp.float32)
        _process(sb_ref, cand_ref, mp, o_ref)

    @pl.when(par == 1)
    def _odd():
        @pl.when(i < steps - 1)
        def _():
            sb_ref[...] = jax.lax.dot_general(
                q, mp, (((1,), (1,)), ((), ())),
                preferred_element_type=jnp.float32)
        _process(sa_ref, cand_ref, mp, o_ref)


@jax.jit
def kernel(input, mempool):
    B, C, H, W = input.shape
    q = jnp.transpose(input, (0, 2, 3, 1)).reshape(-1, C)
    nblk = _N // _BQ
    out = pl.pallas_call(
        _body,
        grid=(nblk + 1,),
        in_specs=[
            pl.BlockSpec((_BQ, _D), lambda i: (jnp.minimum(i, nblk - 1), 0)),
            pl.BlockSpec((_M, _D), lambda i: (0, 0)),
        ],
        out_specs=pl.BlockSpec((_BQ, _D), lambda i: (jnp.maximum(i - 1, 0), 0)),
        out_shape=jax.ShapeDtypeStruct((_N, _D), jnp.float32),
        scratch_shapes=[
            pltpu.VMEM((_BQ, _M), jnp.float32),
            pltpu.VMEM((_BQ, _M), jnp.float32),
            pltpu.VMEM((_BQ, _K * _LANES), jnp.float32),
        ],
    )(q, mempool)
    out = out.reshape(B, H, W, C)
    return jnp.transpose(out, (0, 3, 1, 2))
